# Initial kernel scaffold; baseline (speedup 1.0000x reference)
import jax
import jax.numpy as jnp
from jax.experimental import pallas as pl


def kernel(x, y, edge_index, edge_attr, batch, emb, lin_W, att_src, att_dst, lin_edge_W, att_edge, conv_bias, W1f, b1f, W2f, b2f, W1b, b1b, W2b, b2b):
    raise NotImplementedError("write your pallas kernel here")


# speedup vs baseline: 30.9004x; 30.9004x over previous
import functools

import jax
import jax.numpy as jnp
from jax import lax
from jax.experimental import pallas as pl
from jax.experimental.pallas import tpu as pltpu
from jax.experimental.pallas import tpu_sc as plsc

_N = 10000
_E = 160000
_NT = 10000
_EMB = 128
_DH = 128
_H = 2
_NG = 64
_IN = _EMB + 1

_NC = 2
_NS = 16
_NW = _NC * _NS

_BPAD = 10240
_BPW = _BPAD // _NW

_EPT = _E // _NS
_NPAD = 10240
_NPT = _NPAD // _NS
_CH = 16
_STG = 2560

_RA = 1024
_GA = _NPAD // _RA


def _emb_gather(emb, xpad):
  mesh = plsc.VectorSubcoreMesh(core_axis_name="c", subcore_axis_name="s")

  @functools.partial(
      pl.kernel,
      mesh=mesh,
      compiler_params=pltpu.CompilerParams(needs_layout_passes=False),
      out_type=jax.ShapeDtypeStruct((_BPAD, _EMB), jnp.float32),
      scratch_types=[
          pltpu.VMEM((_BPW,), jnp.int32),
          pltpu.VMEM((_BPW, _EMB), jnp.float32),
          pltpu.SemaphoreType.DMA,
      ],
  )
  def gath(table_hbm, idx_hbm, out_hbm, idx_v, rows_v, sem):
    wid = lax.axis_index("s") * _NC + lax.axis_index("c")
    base = wid * _BPW
    pltpu.sync_copy(idx_hbm.at[pl.ds(base, _BPW)], idx_v)
    pltpu.async_copy(table_hbm.at[idx_v], rows_v, sem).wait()
    pltpu.sync_copy(rows_v, out_hbm.at[pl.ds(base, _BPW)])

  return gath(emb, xpad)


def _ea_mean(ea2d):
  def body(ea_ref, out_ref):
    out_ref[0, 0] = jnp.sum(ea_ref[...]) / float(_E)

  return pl.pallas_call(
      body,
      out_specs=pl.BlockSpec(memory_space=pltpu.SMEM),
      out_shape=jax.ShapeDtypeStruct((1, 1), jnp.float32),
  )(ea2d)


def _project(xy, lin_W, asf, adf, aef, lef, mea):
  def body(xy_ref, lw_ref, as_ref, ad_ref, ae_ref, le_ref, m_ref,
           xs_ref, si_ref, aa_ref):
    xyb = xy_ref[...]
    xs = jnp.dot(xyb, lw_ref[...], preferred_element_type=jnp.float32)
    xs0 = xs[:, :_DH]
    xs1 = xs[:, _DH:]
    asv = as_ref[...]
    adv = ad_ref[...]
    a0s = jnp.sum(xs0 * asv[:, :_DH], axis=1)
    a1s = jnp.sum(xs1 * asv[:, _DH:], axis=1)
    a0d = jnp.sum(xs0 * adv[:, :_DH], axis=1)
    a1d = jnp.sum(xs1 * adv[:, _DH:], axis=1)
    lev = le_ref[...]
    aev = ae_ref[...]
    c0 = jnp.sum(lev[:, :_DH] * aev[:, :_DH])
    c1 = jnp.sum(lev[:, _DH:] * aev[:, _DH:])
    m = m_ref[0, 0]
    al0 = a0s + a0d + m * c0
    al1 = a1s + a1d + m * c1
    al0 = jnp.where(al0 > 0, al0, 0.2 * al0)
    al1 = jnp.where(al1 > 0, al1, 0.2 * al1)
    ex0 = jnp.exp(al0)
    ex1 = jnp.exp(al1)
    xs_ref[0] = xs0
    xs_ref[1] = xs1
    si_ref[0] = ex0[:, None] * xs0
    si_ref[1] = ex1[:, None] * xs1
    aa = jnp.stack([a0s, a1s, a0d, a1d, ex0, ex1,
                    jnp.full((_RA,), c0, jnp.float32),
                    jnp.full((_RA,), c1, jnp.float32)], axis=0)
    aa_ref[...] = aa[None, :, :]

  full = lambda shape: pl.BlockSpec(shape, lambda i: tuple(0 for _ in shape))
  return pl.pallas_call(
      body,
      grid=(_GA,),
      in_specs=[
          pl.BlockSpec((_RA, _IN), lambda i: (i, 0)),
          full((_IN, _H * _DH)),
          full((1, _H * _DH)),
          full((1, _H * _DH)),
          full((1, _H * _DH)),
          full((1, _H * _DH)),
          pl.BlockSpec(memory_space=pltpu.SMEM),
      ],
      out_specs=[
          pl.BlockSpec((_H, _RA, _DH), lambda i: (0, i, 0)),
          pl.BlockSpec((_H, _RA, _DH), lambda i: (0, i, 0)),
          pl.BlockSpec((1, 8, _RA), lambda i: (i, 0, 0)),
      ],
      out_shape=[
          jax.ShapeDtypeStruct((_H, _NPAD, _DH), jnp.float32),
          jax.ShapeDtypeStruct((_H, _NPAD, _DH), jnp.float32),
          jax.ShapeDtypeStruct((_GA, 8, _RA), jnp.float32),
      ],
  )(xy, lin_W, asf, adf, aef, lef, mea)


def _edge_phase(src, dst, ea, xs2, si2, aa3):
  mesh = plsc.VectorSubcoreMesh(core_axis_name="c", subcore_axis_name="s")

  @functools.partial(
      pl.kernel,
      mesh=mesh,
      compiler_params=pltpu.CompilerParams(needs_layout_passes=False),
      out_type=(
          jax.ShapeDtypeStruct((_H, _NPAD, _DH), jnp.float32),
          jax.ShapeDtypeStruct((_GA, _H * _NS, _RA), jnp.float32),
      ),
      scratch_types=[
          pltpu.VMEM((_STG,), jnp.int32),
          pltpu.VMEM((_STG,), jnp.int32),
          pltpu.VMEM((_STG,), jnp.float32),
          pltpu.VMEM((_NPAD,), jnp.float32),
          pltpu.VMEM((_NPAD,), jnp.float32),
          pltpu.VMEM((_NPAD,), jnp.float32),
          pltpu.VMEM((_CH, _DH), jnp.float32),
          pltpu.VMEM((16,), jnp.float32),
          pltpu.VMEM_SHARED((_NPAD, _DH), jnp.float32),
          pltpu.SemaphoreType.DMA,
      ],
  )
  def edge(src_hbm, dst_hbm, ea_hbm, xs_hbm, si_hbm, aa_hbm,
           s_hbm, dp_hbm,
           srcb, dstb, eab, asb, adb, dloc, rowb, cb, sacc, sem):
    c = lax.axis_index("c")
    s = lax.axis_index("s")
    e0 = s * _EPT
    for g in range(_GA):
      pltpu.sync_copy(aa_hbm.at[g, c], asb.at[pl.ds(g * _RA, _RA)])
      pltpu.sync_copy(aa_hbm.at[g, 2 + c], adb.at[pl.ds(g * _RA, _RA)])
    pltpu.sync_copy(aa_hbm.at[0, 6 + c, pl.ds(0, 16)], cb)
    pltpu.sync_copy(si_hbm.at[pl.ds(c * _NPAD + s * _NPT, _NPT)],
                    sacc.at[pl.ds(s * _NPT, _NPT)])

    def zbody(i, carry):
      dloc[pl.ds(i * 16, 16)] = jnp.zeros((16,), jnp.float32)
      return carry

    lax.fori_loop(0, _NPAD // 16, zbody, 0)
    plsc.subcore_barrier()

    cbase = c * _NPAD
    cv = cb[...]

    def body(i, carry):
      off = i * _CH
      sv = srcb[pl.ds(off, _CH)]
      dv = dstb[pl.ds(off, _CH)]
      ev = eab[pl.ds(off, _CH)]
      a = plsc.load_gather(asb, [sv]) + plsc.load_gather(adb, [dv]) + ev * cv
      a = jnp.where(a > 0, a, 0.2 * a)
      ex = jnp.exp(a)
      plsc.addupdate_scatter(dloc, [dv], ex)
      pltpu.async_copy(xs_hbm.at[sv + cbase], rowb, sem).wait()
      for r in range(_CH):
        sc = ex[r]
        for k in range(_DH // 16):
          rowb[r, pl.ds(k * 16, 16)] = rowb[r, pl.ds(k * 16, 16)] * sc
      pltpu.sync_copy(rowb, sacc.at[dv], add=True)
      return carry

    done = 0
    while done < _EPT:
      cnt = min(_STG, _EPT - done)
      pltpu.sync_copy(src_hbm.at[pl.ds(e0 + done, cnt)], srcb.at[pl.ds(0, cnt)])
      pltpu.sync_copy(dst_hbm.at[pl.ds(e0 + done, cnt)], dstb.at[pl.ds(0, cnt)])
      pltpu.sync_copy(ea_hbm.at[pl.ds(e0 + done, cnt)], eab.at[pl.ds(0, cnt)])
      lax.fori_loop(0, cnt // _CH, body, 0)
      done += cnt
    plsc.subcore_barrier()
    for g in range(_GA):
      pltpu.sync_copy(dloc.at[pl.ds(g * _RA, _RA)],
                      dp_hbm.at[g, c * _NS + s])
    pltpu.sync_copy(sacc.at[pl.ds(s * _NPT, _NPT)],
                    s_hbm.at[c, pl.ds(s * _NPT, _NPT)])

  return edge(src, dst, ea, xs2, si2, aa3)


def _pool(s0, s1, dp3, aa3, cbias, batch3):
  def body(s0_ref, s1_ref, dp_ref, aa_ref, cb_ref, bt_ref, g_ref):
    i = pl.program_id(0)
    dpb = dp_ref[0]
    aab = aa_ref[0]
    d0 = jnp.sum(dpb[:_NS], axis=0) + aab[4]
    d1 = jnp.sum(dpb[_NS:], axis=0) + aab[5]
    h = 0.5 * (s0_ref[...] / (d0[:, None] + 1e-16)
               + s1_ref[...] / (d1[:, None] + 1e-16)) + cb_ref[...]
    bt = bt_ref[0]
    oh = (lax.broadcasted_iota(jnp.int32, (_NG, _RA), 0) == bt).astype(
        jnp.float32)

    @pl.when(i == 0)
    def _():
      g_ref[...] = jnp.zeros((_NG, _DH), jnp.float32)

    g_ref[...] += jnp.dot(oh, h, preferred_element_type=jnp.float32)

  return pl.pallas_call(
      body,
      grid=(_GA,),
      in_specs=[
          pl.BlockSpec((_RA, _DH), lambda i: (i, 0)),
          pl.BlockSpec((_RA, _DH), lambda i: (i, 0)),
          pl.BlockSpec((1, _H * _NS, _RA), lambda i: (i, 0, 0)),
          pl.BlockSpec((1, 8, _RA), lambda i: (i, 0, 0)),
          pl.BlockSpec((1, _DH), lambda i: (0, 0)),
          pl.BlockSpec((1, 1, _RA), lambda i: (i, 0, 0)),
      ],
      out_specs=pl.BlockSpec((_NG, _DH), lambda i: (0, 0)),
      out_shape=jax.ShapeDtypeStruct((_NG, _DH), jnp.float32),
  )(s0, s1, dp3, aa3, cbias, batch3)


_NTP = 10240


def _mlps(g, W1f, b1f, W2f, b2f, W1b, b1b, W2b, b2b):
  t = 2048
  gt = _NTP // t

  def body(g_ref, w1f_ref, b1f_ref, w2f_ref, b2f_ref,
           w1b_ref, b1b_ref, w2b_ref, b2b_ref, f_ref, o_ref):
    gg = g_ref[...]
    zf = jnp.maximum(
        jnp.dot(gg, w1f_ref[...], preferred_element_type=jnp.float32)
        + b1f_ref[...], 0.0)
    zb = jnp.maximum(
        jnp.dot(gg, w1b_ref[...], preferred_element_type=jnp.float32)
        + b1b_ref[...], 0.0)
    f_ref[...] = jnp.dot(zf, w2f_ref[...],
                         preferred_element_type=jnp.float32) + b2f_ref[...]
    o_ref[...] = jnp.dot(zb, w2b_ref[...],
                         preferred_element_type=jnp.float32) + b2b_ref[...]

  full = lambda shape: pl.BlockSpec(shape, lambda i: tuple(0 for _ in shape))
  return pl.pallas_call(
      body,
      grid=(gt,),
      in_specs=[
          full((_NG, _DH)),
          full((_DH, _DH)),
          full((1, _DH)),
          pl.BlockSpec((_DH, t), lambda i: (0, i)),
          pl.BlockSpec((1, t), lambda i: (0, i)),
          full((_DH, _DH)),
          full((1, _DH)),
          pl.BlockSpec((_DH, t), lambda i: (0, i)),
          pl.BlockSpec((1, t), lambda i: (0, i)),
      ],
      out_specs=[
          pl.BlockSpec((_NG, t), lambda i: (0, i)),
          pl.BlockSpec((_NG, t), lambda i: (0, i)),
      ],
      out_shape=[
          jax.ShapeDtypeStruct((_NG, _NTP), jnp.float32),
          jax.ShapeDtypeStruct((_NG, _NTP), jnp.float32),
      ],
  )(g, W1f, b1f, W2f, b2f, W1b, b1b, W2b, b2b)


def kernel(x, y, edge_index, edge_attr, batch, emb, lin_W, att_src, att_dst,
           lin_edge_W, att_edge, conv_bias, W1f, b1f, W2f, b2f,
           W1b, b1b, W2b, b2b):
  x = x.astype(jnp.int32)
  src = edge_index[0].astype(jnp.int32)
  dst = edge_index[1].astype(jnp.int32)
  batch = batch.astype(jnp.int32)
  ea = edge_attr[:, 0]

  xpad = jnp.pad(x, (0, _BPAD - _N))
  xe = _emb_gather(emb, xpad)
  ypad = jnp.pad(y, (0, _NPAD - _N))
  xy = jnp.concatenate([xe, ypad[:, None]], axis=1)

  mea = _ea_mean(ea.reshape(160, 1000))

  asf = att_src.reshape(1, _H * _DH)
  adf = att_dst.reshape(1, _H * _DH)
  aef = att_edge.reshape(1, _H * _DH)
  xs3, si3, aa3 = _project(xy, lin_W, asf, adf, aef, lin_edge_W, mea)

  xs2 = xs3.reshape(_H * _NPAD, _DH)
  si2 = si3.reshape(_H * _NPAD, _DH)
  svals, dpart = _edge_phase(src, dst, ea, xs2, si2, aa3)

  g = _pool(svals[0], svals[1], dpart, aa3,
            conv_bias.reshape(1, _DH),
            jnp.pad(batch, (0, _NPAD - _N),
                    constant_values=_NG).reshape(_GA, 1, _RA))

  pw = _NTP - _NT
  f, b = _mlps(g, W1f, b1f.reshape(1, _DH),
               jnp.pad(W2f, ((0, 0), (0, pw))),
               jnp.pad(b2f.reshape(1, _NT), ((0, 0), (0, pw))),
               W1b, b1b.reshape(1, _DH),
               jnp.pad(W2b, ((0, 0), (0, pw))),
               jnp.pad(b2b.reshape(1, _NT), ((0, 0), (0, pw))))
  return (f[:, :_NT], b[:, :_NT])


# speedup vs baseline: 50.8963x; 1.6471x over previous
import functools

import jax
import jax.numpy as jnp
from jax import lax
from jax.experimental import pallas as pl
from jax.experimental.pallas import tpu as pltpu
from jax.experimental.pallas import tpu_sc as plsc

_N = 10000
_E = 160000
_NT = 10000
_EMB = 128
_DH = 128
_H = 2
_NG = 64
_IN = _EMB + 1

_NC = 2
_NS = 16
_NW = _NC * _NS

_BPAD = 10240
_BPW = _BPAD // _NW

_EPT = _E // _NS
_NPAD = 10240
_NPT = _NPAD // _NS
_CH = 16
_STG = 2560

_RA = 1024
_GA = _NPAD // _RA


def _emb_gather(emb, xpad):
  mesh = plsc.VectorSubcoreMesh(core_axis_name="c", subcore_axis_name="s")

  @functools.partial(
      pl.kernel,
      mesh=mesh,
      compiler_params=pltpu.CompilerParams(needs_layout_passes=False),
      out_type=jax.ShapeDtypeStruct((_BPAD, _EMB), jnp.float32),
      scratch_types=[
          pltpu.VMEM((_BPW,), jnp.int32),
          pltpu.VMEM((_BPW, _EMB), jnp.float32),
          pltpu.SemaphoreType.DMA,
      ],
  )
  def gath(table_hbm, idx_hbm, out_hbm, idx_v, rows_v, sem):
    wid = lax.axis_index("s") * _NC + lax.axis_index("c")
    base = wid * _BPW
    pltpu.sync_copy(idx_hbm.at[pl.ds(base, _BPW)], idx_v)
    pltpu.async_copy(table_hbm.at[idx_v], rows_v, sem).wait()
    pltpu.sync_copy(rows_v, out_hbm.at[pl.ds(base, _BPW)])

  return gath(emb, xpad)


def _ea_mean(ea2d):
  def body(ea_ref, out_ref):
    out_ref[0, 0] = jnp.sum(ea_ref[...]) / float(_E)

  return pl.pallas_call(
      body,
      out_specs=pl.BlockSpec(memory_space=pltpu.SMEM),
      out_shape=jax.ShapeDtypeStruct((1, 1), jnp.float32),
  )(ea2d)


def _project(xy, lin_W, asf, adf, aef, lef, mea):
  def body(xy_ref, lw_ref, as_ref, ad_ref, ae_ref, le_ref, m_ref,
           xs_ref, si_ref, aa_ref):
    xyb = xy_ref[...]
    xs = jnp.dot(xyb, lw_ref[...], preferred_element_type=jnp.float32)
    xs0 = xs[:, :_DH]
    xs1 = xs[:, _DH:]
    asv = as_ref[...]
    adv = ad_ref[...]
    a0s = jnp.sum(xs0 * asv[:, :_DH], axis=1)
    a1s = jnp.sum(xs1 * asv[:, _DH:], axis=1)
    a0d = jnp.sum(xs0 * adv[:, :_DH], axis=1)
    a1d = jnp.sum(xs1 * adv[:, _DH:], axis=1)
    lev = le_ref[...]
    aev = ae_ref[...]
    c0 = jnp.sum(lev[:, :_DH] * aev[:, :_DH])
    c1 = jnp.sum(lev[:, _DH:] * aev[:, _DH:])
    m = m_ref[0, 0]
    al0 = a0s + a0d + m * c0
    al1 = a1s + a1d + m * c1
    al0 = jnp.where(al0 > 0, al0, 0.2 * al0)
    al1 = jnp.where(al1 > 0, al1, 0.2 * al1)
    ex0 = jnp.exp(al0)
    ex1 = jnp.exp(al1)
    xs_ref[0] = xs0
    xs_ref[1] = xs1
    si_ref[0] = ex0[:, None] * xs0
    si_ref[1] = ex1[:, None] * xs1
    aa = jnp.stack([a0s, a1s, a0d, a1d, ex0, ex1,
                    jnp.full((_RA,), c0, jnp.float32),
                    jnp.full((_RA,), c1, jnp.float32)], axis=0)
    aa_ref[...] = aa[None, :, :]

  full = lambda shape: pl.BlockSpec(shape, lambda i: tuple(0 for _ in shape))
  return pl.pallas_call(
      body,
      grid=(_GA,),
      in_specs=[
          pl.BlockSpec((_RA, _IN), lambda i: (i, 0)),
          full((_IN, _H * _DH)),
          full((1, _H * _DH)),
          full((1, _H * _DH)),
          full((1, _H * _DH)),
          full((1, _H * _DH)),
          pl.BlockSpec(memory_space=pltpu.SMEM),
      ],
      out_specs=[
          pl.BlockSpec((_H, _RA, _DH), lambda i: (0, i, 0)),
          pl.BlockSpec((_H, _RA, _DH), lambda i: (0, i, 0)),
          pl.BlockSpec((1, 8, _RA), lambda i: (i, 0, 0)),
      ],
      out_shape=[
          jax.ShapeDtypeStruct((_H, _NPAD, _DH), jnp.float32),
          jax.ShapeDtypeStruct((_H, _NPAD, _DH), jnp.float32),
          jax.ShapeDtypeStruct((_GA, 8, _RA), jnp.float32),
      ],
  )(xy, lin_W, asf, adf, aef, lef, mea)


def _edge_phase(src, dst, ea, xs2, si2, aa3):
  mesh = plsc.VectorSubcoreMesh(core_axis_name="c", subcore_axis_name="s")

  @functools.partial(
      pl.kernel,
      mesh=mesh,
      compiler_params=pltpu.CompilerParams(needs_layout_passes=False),
      out_type=(
          jax.ShapeDtypeStruct((_H, _NPAD, _DH), jnp.float32),
          jax.ShapeDtypeStruct((_GA, _H * _NS, _RA), jnp.float32),
      ),
      scratch_types=[
          pltpu.VMEM((_STG,), jnp.int32),
          pltpu.VMEM((_STG,), jnp.int32),
          pltpu.VMEM((_STG,), jnp.float32),
          pltpu.VMEM((_NPAD,), jnp.float32),
          pltpu.VMEM((_NPAD,), jnp.float32),
          pltpu.VMEM((_NPAD,), jnp.float32),
          pltpu.VMEM((_CH, _DH), jnp.float32),
          pltpu.VMEM((_CH, _DH), jnp.float32),
          pltpu.VMEM((16,), jnp.float32),
          pltpu.VMEM_SHARED((_NPAD, _DH), jnp.float32),
          pltpu.SemaphoreType.DMA,
          pltpu.SemaphoreType.DMA,
      ],
  )
  def edge(src_hbm, dst_hbm, ea_hbm, xs_hbm, si_hbm, aa_hbm,
           s_hbm, dp_hbm,
           srcb, dstb, eab, asb, adb, dloc, rowb0, rowb1, cb, sacc,
           sem0, sem1):
    c = lax.axis_index("c")
    s = lax.axis_index("s")
    e0 = s * _EPT
    for g in range(_GA):
      pltpu.sync_copy(aa_hbm.at[g, c], asb.at[pl.ds(g * _RA, _RA)])
      pltpu.sync_copy(aa_hbm.at[g, 2 + c], adb.at[pl.ds(g * _RA, _RA)])
    pltpu.sync_copy(aa_hbm.at[0, 6 + c, pl.ds(0, 16)], cb)
    pltpu.sync_copy(si_hbm.at[pl.ds(c * _NPAD + s * _NPT, _NPT)],
                    sacc.at[pl.ds(s * _NPT, _NPT)])

    def zbody(i, carry):
      dloc[pl.ds(i * 16, 16)] = jnp.zeros((16,), jnp.float32)
      return carry

    lax.fori_loop(0, _NPAD // 16, zbody, 0)
    plsc.subcore_barrier()

    cbase = c * _NPAD
    cv = cb[...]
    bufs = ((rowb0, sem0), (rowb1, sem1))

    def start_gather(i, cnt, buf, sem):
      off = jnp.where(i < cnt, i, 0) * _CH
      sv = srcb[pl.ds(off, _CH)]
      return pltpu.async_copy(xs_hbm.at[sv + cbase], buf, sem)

    def process(i, cnt, buf, sem):
      off = i * _CH
      sv = srcb[pl.ds(off, _CH)]
      dv = dstb[pl.ds(off, _CH)]
      ev = eab[pl.ds(off, _CH)]
      a = plsc.load_gather(asb, [sv]) + plsc.load_gather(adb, [dv]) + ev * cv
      a = jnp.where(a > 0, a, 0.2 * a)
      ex = jnp.exp(a)
      plsc.addupdate_scatter(dloc, [dv], ex)
      pltpu.make_async_copy(xs_hbm.at[sv + cbase], buf, sem).wait()
      for r in range(_CH):
        sc = ex[r]
        for k in range(_DH // 16):
          buf[r, pl.ds(k * 16, 16)] = buf[r, pl.ds(k * 16, 16)] * sc
      pltpu.sync_copy(buf, sacc.at[dv], add=True)

    done = 0
    while done < _EPT:
      cnt = min(_STG, _EPT - done)
      nch = cnt // _CH
      pltpu.sync_copy(src_hbm.at[pl.ds(e0 + done, cnt)], srcb.at[pl.ds(0, cnt)])
      pltpu.sync_copy(dst_hbm.at[pl.ds(e0 + done, cnt)], dstb.at[pl.ds(0, cnt)])
      pltpu.sync_copy(ea_hbm.at[pl.ds(e0 + done, cnt)], eab.at[pl.ds(0, cnt)])
      start_gather(0, nch, *bufs[0])

      def pair(j, carry):
        i0 = 2 * j
        start_gather(i0 + 1, nch, *bufs[1])
        process(i0, nch, *bufs[0])
        start_gather(i0 + 2, nch, *bufs[0])
        process(i0 + 1, nch, *bufs[1])
        return carry

      lax.fori_loop(0, nch // 2, pair, 0)
      if nch % 2:
        process(nch - 1, nch, *bufs[0])
      else:
        pltpu.make_async_copy(xs_hbm.at[srcb[pl.ds(0, _CH)] + cbase],
                              bufs[0][0], bufs[0][1]).wait()
      done += cnt
    plsc.subcore_barrier()
    for g in range(_GA):
      pltpu.sync_copy(dloc.at[pl.ds(g * _RA, _RA)],
                      dp_hbm.at[g, c * _NS + s])
    pltpu.sync_copy(sacc.at[pl.ds(s * _NPT, _NPT)],
                    s_hbm.at[c, pl.ds(s * _NPT, _NPT)])

  return edge(src, dst, ea, xs2, si2, aa3)


def _pool(s0, s1, dp3, aa3, cbias, batch3):
  def body(s0_ref, s1_ref, dp_ref, aa_ref, cb_ref, bt_ref, g_ref):
    i = pl.program_id(0)
    dpb = dp_ref[0]
    aab = aa_ref[0]
    d0 = jnp.sum(dpb[:_NS], axis=0) + aab[4]
    d1 = jnp.sum(dpb[_NS:], axis=0) + aab[5]
    h = 0.5 * (s0_ref[...] / (d0[:, None] + 1e-16)
               + s1_ref[...] / (d1[:, None] + 1e-16)) + cb_ref[...]
    bt = bt_ref[0]
    oh = (lax.broadcasted_iota(jnp.int32, (_NG, _RA), 0) == bt).astype(
        jnp.float32)

    @pl.when(i == 0)
    def _():
      g_ref[...] = jnp.zeros((_NG, _DH), jnp.float32)

    g_ref[...] += jnp.dot(oh, h, preferred_element_type=jnp.float32)

  return pl.pallas_call(
      body,
      grid=(_GA,),
      in_specs=[
          pl.BlockSpec((_RA, _DH), lambda i: (i, 0)),
          pl.BlockSpec((_RA, _DH), lambda i: (i, 0)),
          pl.BlockSpec((1, _H * _NS, _RA), lambda i: (i, 0, 0)),
          pl.BlockSpec((1, 8, _RA), lambda i: (i, 0, 0)),
          pl.BlockSpec((1, _DH), lambda i: (0, 0)),
          pl.BlockSpec((1, 1, _RA), lambda i: (i, 0, 0)),
      ],
      out_specs=pl.BlockSpec((_NG, _DH), lambda i: (0, 0)),
      out_shape=jax.ShapeDtypeStruct((_NG, _DH), jnp.float32),
  )(s0, s1, dp3, aa3, cbias, batch3)


_NTP = 10240


def _mlps(g, W1f, b1f, W2f, b2f, W1b, b1b, W2b, b2b):
  t = 2048
  gt = _NTP // t

  def body(g_ref, w1f_ref, b1f_ref, w2f_ref, b2f_ref,
           w1b_ref, b1b_ref, w2b_ref, b2b_ref, f_ref, o_ref):
    gg = g_ref[...]
    zf = jnp.maximum(
        jnp.dot(gg, w1f_ref[...], preferred_element_type=jnp.float32)
        + b1f_ref[...], 0.0)
    zb = jnp.maximum(
        jnp.dot(gg, w1b_ref[...], preferred_element_type=jnp.float32)
        + b1b_ref[...], 0.0)
    f_ref[...] = jnp.dot(zf, w2f_ref[...],
                         preferred_element_type=jnp.float32) + b2f_ref[...]
    o_ref[...] = jnp.dot(zb, w2b_ref[...],
                         preferred_element_type=jnp.float32) + b2b_ref[...]

  full = lambda shape: pl.BlockSpec(shape, lambda i: tuple(0 for _ in shape))
  return pl.pallas_call(
      body,
      grid=(gt,),
      in_specs=[
          full((_NG, _DH)),
          full((_DH, _DH)),
          full((1, _DH)),
          pl.BlockSpec((_DH, t), lambda i: (0, i)),
          pl.BlockSpec((1, t), lambda i: (0, i)),
          full((_DH, _DH)),
          full((1, _DH)),
          pl.BlockSpec((_DH, t), lambda i: (0, i)),
          pl.BlockSpec((1, t), lambda i: (0, i)),
      ],
      out_specs=[
          pl.BlockSpec((_NG, t), lambda i: (0, i)),
          pl.BlockSpec((_NG, t), lambda i: (0, i)),
      ],
      out_shape=[
          jax.ShapeDtypeStruct((_NG, _NTP), jnp.float32),
          jax.ShapeDtypeStruct((_NG, _NTP), jnp.float32),
      ],
  )(g, W1f, b1f, W2f, b2f, W1b, b1b, W2b, b2b)


def kernel(x, y, edge_index, edge_attr, batch, emb, lin_W, att_src, att_dst,
           lin_edge_W, att_edge, conv_bias, W1f, b1f, W2f, b2f,
           W1b, b1b, W2b, b2b):
  x = x.astype(jnp.int32)
  src = edge_index[0].astype(jnp.int32)
  dst = edge_index[1].astype(jnp.int32)
  batch = batch.astype(jnp.int32)
  ea = edge_attr[:, 0]

  xpad = jnp.pad(x, (0, _BPAD - _N))
  xe = _emb_gather(emb, xpad)
  ypad = jnp.pad(y, (0, _NPAD - _N))
  xy = jnp.concatenate([xe, ypad[:, None]], axis=1)

  mea = _ea_mean(ea.reshape(160, 1000))

  asf = att_src.reshape(1, _H * _DH)
  adf = att_dst.reshape(1, _H * _DH)
  aef = att_edge.reshape(1, _H * _DH)
  xs3, si3, aa3 = _project(xy, lin_W, asf, adf, aef, lin_edge_W, mea)

  xs2 = xs3.reshape(_H * _NPAD, _DH)
  si2 = si3.reshape(_H * _NPAD, _DH)
  svals, dpart = _edge_phase(src, dst, ea, xs2, si2, aa3)

  g = _pool(svals[0], svals[1], dpart, aa3,
            conv_bias.reshape(1, _DH),
            jnp.pad(batch, (0, _NPAD - _N),
                    constant_values=_NG).reshape(_GA, 1, _RA))

  pw = _NTP - _NT
  f, b = _mlps(g, W1f, b1f.reshape(1, _DH),
               jnp.pad(W2f, ((0, 0), (0, pw))),
               jnp.pad(b2f.reshape(1, _NT), ((0, 0), (0, pw))),
               W1b, b1b.reshape(1, _DH),
               jnp.pad(W2b, ((0, 0), (0, pw))),
               jnp.pad(b2b.reshape(1, _NT), ((0, 0), (0, pw))))
  return (f[:, :_NT], b[:, :_NT])
